# baseline (speedup 1.0000x reference)
import jax
import jax.numpy as jnp
from jax import lax
from jax.experimental import pallas as pl
from jax.experimental.pallas import tpu as pltpu
from jax.experimental.pallas import tpu_sc as plsc

T = 2048
D = 1024
DFF = 2048
E = 8
K = 2

BT = 256
G_MAX = T * K // BT + E
N_PAD = G_MAX * BT

NC = 2
NS = 16
NW = NC * NS
TPW = T // NW
CH = 32


def _router_body(x_ref, wg_ref, p0_ref, p1_ref, be_ref, nv_ref,
                 g0_ref, g1_ref):
    logits = jnp.dot(x_ref[...], wg_ref[...],
                     preferred_element_type=jnp.float32)
    m0 = jnp.max(logits, axis=1)
    i0 = jnp.argmax(logits, axis=1).astype(jnp.int32)
    lane = lax.broadcasted_iota(jnp.int32, (T, E), 1)
    masked = jnp.where(lane == i0[:, None], -jnp.inf, logits)
    m1 = jnp.max(masked, axis=1)
    i1 = jnp.argmax(masked, axis=1).astype(jnp.int32)
    r = jnp.exp(m1 - m0)
    g0_ref[...] = jnp.broadcast_to((1.0 / (1.0 + r))[:, None], (T, 128))
    g1_ref[...] = jnp.broadcast_to((r / (1.0 + r))[:, None], (T, 128))

    e_all = jnp.concatenate([i0, i1])
    erow = jnp.arange(E, dtype=jnp.int32)
    hot = (e_all[None, :] == erow[:, None]).astype(jnp.int32)
    csum = hot
    shift = 1
    while shift < T * K:
        z = jnp.zeros((E, shift), jnp.int32)
        csum = csum + jnp.concatenate([z, csum[:, :-shift]], axis=1)
        shift *= 2
    rank = jnp.sum(hot * csum, axis=0) - 1
    counts = csum[:, -1]
    nblk = (counts + BT - 1) // BT
    tri = (erow[:, None] <= erow[None, :]).astype(jnp.float32)
    cs = jnp.dot(nblk.astype(jnp.float32), tri,
                 preferred_element_type=jnp.float32).astype(jnp.int32)
    total = cs[E - 1]
    base_rows = (cs - nblk) * BT
    pos = jnp.sum(hot * base_rows[:, None], axis=0) + rank
    p0_ref[...] = pos[:T]
    p1_ref[...] = pos[T:]
    gcol = lax.broadcasted_iota(jnp.int32, (G_MAX, E), 0)
    csb = jnp.broadcast_to(cs[None, :], (G_MAX, E))
    be_ref[...] = jnp.sum(
        ((gcol >= csb) & (csb < total)).astype(jnp.int32), axis=1)
    nv_ref[...] = jnp.broadcast_to(total, (1,))


def _router(x, wg):
    return pl.pallas_call(
        _router_body,
        out_shape=(
            jax.ShapeDtypeStruct((T,), jnp.int32),
            jax.ShapeDtypeStruct((T,), jnp.int32),
            jax.ShapeDtypeStruct((G_MAX,), jnp.int32),
            jax.ShapeDtypeStruct((1,), jnp.int32),
            jax.ShapeDtypeStruct((T, 128), jnp.float32),
            jax.ShapeDtypeStruct((T, 128), jnp.float32),
        ),
    )(x, wg)


def _dispatch_body(x_hbm, p0_hbm, p1_hbm, xs_hbm, xrows, idx0, idx1, sem):
    wid = lax.axis_index("s") * NC + lax.axis_index("c")
    base = wid * TPW
    pltpu.sync_copy(x_hbm.at[pl.ds(base, TPW)], xrows)
    pltpu.sync_copy(p0_hbm.at[pl.ds(base, TPW)], idx0)
    pltpu.sync_copy(p1_hbm.at[pl.ds(base, TPW)], idx1)
    c0 = pltpu.async_copy(xrows, xs_hbm.at[idx0], sem)
    c1 = pltpu.async_copy(xrows, xs_hbm.at[idx1], sem)
    c0.wait()
    c1.wait()


def _dispatch(x, pos0, pos1):
    mesh = plsc.VectorSubcoreMesh(core_axis_name="c", subcore_axis_name="s")
    return pl.kernel(
        _dispatch_body,
        out_type=jax.ShapeDtypeStruct((N_PAD, D), jnp.float32),
        mesh=mesh,
        scratch_types=[
            pltpu.VMEM((TPW, D), jnp.float32),
            pltpu.VMEM((TPW,), jnp.int32),
            pltpu.VMEM((TPW,), jnp.int32),
            pltpu.SemaphoreType.DMA,
        ],
    )(x, pos0, pos1)


FF = 4
BF = DFF // FF


def _zz(g, f):
    return jnp.where(g % 2 == 0, f, FF - 1 - f)


def _gmm_body(be_ref, nv_ref, x_ref, wg_ref, bg_ref, wu_ref, bu_ref,
              wd_ref, bd_ref, o_ref, acc_ref):
    f = pl.program_id(1)

    @pl.when(pl.program_id(0) < nv_ref[0])
    def _():
        xb = x_ref[...].astype(jnp.bfloat16)
        h = jnp.dot(xb, wg_ref[0].astype(jnp.bfloat16),
                    preferred_element_type=jnp.float32) + bg_ref[0]
        u = jnp.dot(xb, wu_ref[0].astype(jnp.bfloat16),
                    preferred_element_type=jnp.float32) + bu_ref[0]
        act = (h * jax.nn.sigmoid(h) * u).astype(jnp.bfloat16)
        part = jnp.dot(act, wd_ref[0].astype(jnp.bfloat16),
                       preferred_element_type=jnp.float32)

        @pl.when(f == 0)
        def _():
            acc_ref[...] = part

        @pl.when(f > 0)
        def _():
            acc_ref[...] += part

        @pl.when(f == FF - 1)
        def _():
            o_ref[...] = acc_ref[...] + bd_ref[0]


def _experts(xs, W_gate, b_gate, W_up, b_up, W_down, b_down,
             block_expert, nvalid):
    grid_spec = pltpu.PrefetchScalarGridSpec(
        num_scalar_prefetch=2,
        grid=(G_MAX, FF),
        in_specs=[
            pl.BlockSpec((BT, D), lambda g, f, be, nv: (g, 0)),
            pl.BlockSpec((1, D, BF),
                         lambda g, f, be, nv: (be[g], 0, _zz(g, f))),
            pl.BlockSpec((1, 1, BF),
                         lambda g, f, be, nv: (be[g], 0, _zz(g, f))),
            pl.BlockSpec((1, D, BF),
                         lambda g, f, be, nv: (be[g], 0, _zz(g, f))),
            pl.BlockSpec((1, 1, BF),
                         lambda g, f, be, nv: (be[g], 0, _zz(g, f))),
            pl.BlockSpec((1, BF, D),
                         lambda g, f, be, nv: (be[g], _zz(g, f), 0)),
            pl.BlockSpec((1, 1, D), lambda g, f, be, nv: (be[g], 0, 0)),
        ],
        out_specs=pl.BlockSpec((BT, D), lambda g, f, be, nv: (g, 0)),
        scratch_shapes=[pltpu.VMEM((BT, D), jnp.float32)],
    )
    return pl.pallas_call(
        _gmm_body,
        grid_spec=grid_spec,
        out_shape=jax.ShapeDtypeStruct((N_PAD, D), jnp.float32),
        compiler_params=pltpu.CompilerParams(
            dimension_semantics=("arbitrary", "arbitrary"),
            vmem_limit_bytes=128 * 1024 * 1024,
        ),
    )(block_expert, nvalid, xs,
      W_gate, b_gate.reshape(E, 1, DFF),
      W_up, b_up.reshape(E, 1, DFF),
      W_down, b_down.reshape(E, 1, D))


def _combine_body(ys_hbm, p0_hbm, p1_hbm, g0_hbm, g1_hbm, y_hbm,
                  rows0, rows1, g0v, g1v, idx0, idx1, sem):
    wid = lax.axis_index("s") * NC + lax.axis_index("c")
    for c in range(TPW // CH):
        base = wid * TPW + c * CH
        pltpu.sync_copy(p0_hbm.at[pl.ds(base, CH)], idx0)
        pltpu.sync_copy(p1_hbm.at[pl.ds(base, CH)], idx1)
        pltpu.sync_copy(g0_hbm.at[pl.ds(base, CH)], g0v)
        pltpu.sync_copy(g1_hbm.at[pl.ds(base, CH)], g1v)
        c0 = pltpu.async_copy(ys_hbm.at[idx0], rows0, sem)
        c1 = pltpu.async_copy(ys_hbm.at[idx1], rows1, sem)
        c0.wait()
        c1.wait()
        for j in range(CH):
            ga = g0v[j, 0:16]
            gb = g1v[j, 0:16]

            def body(i, _, ga=ga, gb=gb, j=j):
                for u in range(8):
                    s = pl.ds(i * 128 + u * 16, 16)
                    rows0[j, s] = ga * rows0[j, s] + gb * rows1[j, s]
                return 0

            lax.fori_loop(0, D // 128, body, 0)
        pltpu.sync_copy(rows0, y_hbm.at[pl.ds(base, CH)])


def _combine(ys, pos0, pos1, G0, G1):
    mesh = plsc.VectorSubcoreMesh(core_axis_name="c", subcore_axis_name="s")
    return pl.kernel(
        _combine_body,
        out_type=jax.ShapeDtypeStruct((T, D), jnp.float32),
        mesh=mesh,
        scratch_types=[
            pltpu.VMEM((CH, D), jnp.float32),
            pltpu.VMEM((CH, D), jnp.float32),
            pltpu.VMEM((CH, 128), jnp.float32),
            pltpu.VMEM((CH, 128), jnp.float32),
            pltpu.VMEM((CH,), jnp.int32),
            pltpu.VMEM((CH,), jnp.int32),
            pltpu.SemaphoreType.DMA,
        ],
    )(ys, pos0, pos1, G0, G1)


def kernel(x, wg, W_gate, b_gate, W_up, b_up, W_down, b_down):
    pos0, pos1, block_expert, nvalid, G0, G1 = _router(x, wg)
    xs = _dispatch(x, pos0, pos1)
    ys = _experts(xs, W_gate, b_gate, W_up, b_up, W_down, b_down,
                  block_expert, nvalid)
    return _combine(ys, pos0, pos1, G0, G1)


# speedup vs baseline: 1.4367x; 1.4367x over previous
import jax
import jax.numpy as jnp
from jax import lax
from jax.experimental import pallas as pl
from jax.experimental.pallas import tpu as pltpu
from jax.experimental.pallas import tpu_sc as plsc

T = 2048
D = 1024
DFF = 2048
E = 8
K = 2

BT = 256
G_MAX = T * K // BT + E
N_PAD = G_MAX * BT

NC = 2
NS = 16
NW = NC * NS
TPW = T // NW
CH = 32


HMASK = -65536


def _pack_rows(y):
    half = y.shape[1] // 2
    a = y[:, :half].astype(jnp.bfloat16).astype(jnp.float32)
    b = y[:, half:].astype(jnp.bfloat16).astype(jnp.float32)
    ia = lax.bitcast_convert_type(a, jnp.int32)
    ib = lax.bitcast_convert_type(b, jnp.int32)
    return lax.shift_right_logical(ia, 16) | (ib & HMASK)


def _unpack_rows(w):
    a = lax.bitcast_convert_type(lax.shift_left(w, 16), jnp.float32)
    b = lax.bitcast_convert_type(w & HMASK, jnp.float32)
    return jnp.concatenate([a, b], axis=1).astype(jnp.bfloat16)


def _router_body(x_ref, wg_ref, p0_ref, p1_ref, be_ref, nv_ref,
                 g0_ref, g1_ref, xp_ref):
    xp_ref[...] = _pack_rows(x_ref[...])
    logits = jnp.dot(x_ref[...], wg_ref[...],
                     preferred_element_type=jnp.float32)
    m0 = jnp.max(logits, axis=1)
    i0 = jnp.argmax(logits, axis=1).astype(jnp.int32)
    lane = lax.broadcasted_iota(jnp.int32, (T, E), 1)
    masked = jnp.where(lane == i0[:, None], -jnp.inf, logits)
    m1 = jnp.max(masked, axis=1)
    i1 = jnp.argmax(masked, axis=1).astype(jnp.int32)
    r = jnp.exp(m1 - m0)
    g0_ref[...] = jnp.broadcast_to((1.0 / (1.0 + r))[:, None], (T, 128))
    g1_ref[...] = jnp.broadcast_to((r / (1.0 + r))[:, None], (T, 128))

    e_all = jnp.concatenate([i0, i1])
    erow = jnp.arange(E, dtype=jnp.int32)
    hot = (e_all[None, :] == erow[:, None]).astype(jnp.int32)
    csum = hot
    shift = 1
    while shift < T * K:
        z = jnp.zeros((E, shift), jnp.int32)
        csum = csum + jnp.concatenate([z, csum[:, :-shift]], axis=1)
        shift *= 2
    rank = jnp.sum(hot * csum, axis=0) - 1
    counts = csum[:, -1]
    nblk = (counts + BT - 1) // BT
    tri = (erow[:, None] <= erow[None, :]).astype(jnp.float32)
    cs = jnp.dot(nblk.astype(jnp.float32), tri,
                 preferred_element_type=jnp.float32).astype(jnp.int32)
    total = cs[E - 1]
    base_rows = (cs - nblk) * BT
    pos = jnp.sum(hot * base_rows[:, None], axis=0) + rank
    p0_ref[...] = pos[:T]
    p1_ref[...] = pos[T:]
    gcol = lax.broadcasted_iota(jnp.int32, (G_MAX, E), 0)
    csb = jnp.broadcast_to(cs[None, :], (G_MAX, E))
    be_ref[...] = jnp.sum(
        ((gcol >= csb) & (csb < total)).astype(jnp.int32), axis=1)
    nv_ref[...] = jnp.broadcast_to(total, (1,))


def _router(x, wg):
    return pl.pallas_call(
        _router_body,
        out_shape=(
            jax.ShapeDtypeStruct((T,), jnp.int32),
            jax.ShapeDtypeStruct((T,), jnp.int32),
            jax.ShapeDtypeStruct((G_MAX,), jnp.int32),
            jax.ShapeDtypeStruct((1,), jnp.int32),
            jax.ShapeDtypeStruct((T, 128), jnp.float32),
            jax.ShapeDtypeStruct((T, 128), jnp.float32),
            jax.ShapeDtypeStruct((T, D // 2), jnp.int32),
        ),
    )(x, wg)


def _dispatch_body(x_hbm, p0_hbm, p1_hbm, xs_hbm, xrows, idx0, idx1, sem):
    wid = lax.axis_index("s") * NC + lax.axis_index("c")
    base = wid * TPW
    pltpu.sync_copy(x_hbm.at[pl.ds(base, TPW)], xrows)
    pltpu.sync_copy(p0_hbm.at[pl.ds(base, TPW)], idx0)
    pltpu.sync_copy(p1_hbm.at[pl.ds(base, TPW)], idx1)
    c0 = pltpu.async_copy(xrows, xs_hbm.at[idx0], sem)
    c1 = pltpu.async_copy(xrows, xs_hbm.at[idx1], sem)
    c0.wait()
    c1.wait()


def _dispatch(x, pos0, pos1):
    mesh = plsc.VectorSubcoreMesh(core_axis_name="c", subcore_axis_name="s")
    return pl.kernel(
        _dispatch_body,
        out_type=jax.ShapeDtypeStruct((N_PAD, D // 2), jnp.int32),
        mesh=mesh,
        scratch_types=[
            pltpu.VMEM((TPW, D // 2), jnp.int32),
            pltpu.VMEM((TPW,), jnp.int32),
            pltpu.VMEM((TPW,), jnp.int32),
            pltpu.SemaphoreType.DMA,
        ],
    )(x, pos0, pos1)


def _gmm_body(be_ref, nv_ref, x_ref, wg_ref, bg_ref, wu_ref, bu_ref,
              wd_ref, bd_ref, o_ref):
    @pl.when(pl.program_id(0) < nv_ref[0])
    def _():
        xb = _unpack_rows(x_ref[...])
        h = jnp.dot(xb, wg_ref[0].astype(jnp.bfloat16),
                    preferred_element_type=jnp.float32) + bg_ref[0]
        u = jnp.dot(xb, wu_ref[0].astype(jnp.bfloat16),
                    preferred_element_type=jnp.float32) + bu_ref[0]
        act = (h * jax.nn.sigmoid(h) * u).astype(jnp.bfloat16)
        y = jnp.dot(act, wd_ref[0].astype(jnp.bfloat16),
                    preferred_element_type=jnp.float32) + bd_ref[0]
        o_ref[...] = _pack_rows(y)


def _experts(xs, W_gate, b_gate, W_up, b_up, W_down, b_down,
             block_expert, nvalid):
    grid_spec = pltpu.PrefetchScalarGridSpec(
        num_scalar_prefetch=2,
        grid=(G_MAX,),
        in_specs=[
            pl.BlockSpec((BT, D // 2),
                         lambda g, be, nv: (jnp.minimum(g, nv[0] - 1), 0)),
            pl.BlockSpec((1, D, DFF), lambda g, be, nv: (be[g], 0, 0)),
            pl.BlockSpec((1, 1, DFF), lambda g, be, nv: (be[g], 0, 0)),
            pl.BlockSpec((1, D, DFF), lambda g, be, nv: (be[g], 0, 0)),
            pl.BlockSpec((1, 1, DFF), lambda g, be, nv: (be[g], 0, 0)),
            pl.BlockSpec((1, DFF, D), lambda g, be, nv: (be[g], 0, 0)),
            pl.BlockSpec((1, 1, D), lambda g, be, nv: (be[g], 0, 0)),
        ],
        out_specs=pl.BlockSpec((BT, D // 2),
                               lambda g, be, nv: (jnp.minimum(g, nv[0]), 0)),
    )
    return pl.pallas_call(
        _gmm_body,
        grid_spec=grid_spec,
        out_shape=jax.ShapeDtypeStruct((N_PAD, D // 2), jnp.int32),
        compiler_params=pltpu.CompilerParams(
            dimension_semantics=("arbitrary",),
            vmem_limit_bytes=128 * 1024 * 1024,
        ),
    )(block_expert, nvalid, xs,
      W_gate, b_gate.reshape(E, 1, DFF),
      W_up, b_up.reshape(E, 1, DFF),
      W_down, b_down.reshape(E, 1, D))


def _combine_body(ys_hbm, p0_hbm, p1_hbm, g0_hbm, g1_hbm, y_hbm,
                  rows0, rows1, yout, g0v, g1v, idx0, idx1, sem):
    wid = lax.axis_index("s") * NC + lax.axis_index("c")
    half = D // 2
    for c in range(TPW // CH):
        base = wid * TPW + c * CH
        pltpu.sync_copy(p0_hbm.at[pl.ds(base, CH)], idx0)
        pltpu.sync_copy(p1_hbm.at[pl.ds(base, CH)], idx1)
        pltpu.sync_copy(g0_hbm.at[pl.ds(base, CH)], g0v)
        pltpu.sync_copy(g1_hbm.at[pl.ds(base, CH)], g1v)
        c0 = pltpu.async_copy(ys_hbm.at[idx0], rows0, sem)
        c1 = pltpu.async_copy(ys_hbm.at[idx1], rows1, sem)
        c0.wait()
        c1.wait()
        for j in range(CH):
            ga = g0v[j, 0:16]
            gb = g1v[j, 0:16]

            def body(i, _, ga=ga, gb=gb, j=j):
                for u in range(4):
                    s = pl.ds(i * 64 + u * 16, 16)
                    w0 = rows0[j, s]
                    w1 = rows1[j, s]
                    a0 = plsc.bitcast(lax.shift_left(w0, 16), jnp.float32)
                    a1 = plsc.bitcast(lax.shift_left(w1, 16), jnp.float32)
                    b0 = plsc.bitcast(w0 & HMASK, jnp.float32)
                    b1 = plsc.bitcast(w1 & HMASK, jnp.float32)
                    lo = pl.ds(i * 64 + u * 16, 16)
                    hi = pl.ds(half + i * 64 + u * 16, 16)
                    yout[j, lo] = ga * a0 + gb * a1
                    yout[j, hi] = ga * b0 + gb * b1
                return 0

            lax.fori_loop(0, half // 64, body, 0)
        pltpu.sync_copy(yout, y_hbm.at[pl.ds(base, CH)])


def _combine(ys, pos0, pos1, G0, G1):
    mesh = plsc.VectorSubcoreMesh(core_axis_name="c", subcore_axis_name="s")
    return pl.kernel(
        _combine_body,
        out_type=jax.ShapeDtypeStruct((T, D), jnp.float32),
        mesh=mesh,
        compiler_params=pltpu.CompilerParams(needs_layout_passes=False),
        scratch_types=[
            pltpu.VMEM((CH, D // 2), jnp.int32),
            pltpu.VMEM((CH, D // 2), jnp.int32),
            pltpu.VMEM((CH, D), jnp.float32),
            pltpu.VMEM((CH, 128), jnp.float32),
            pltpu.VMEM((CH, 128), jnp.float32),
            pltpu.VMEM((CH,), jnp.int32),
            pltpu.VMEM((CH,), jnp.int32),
            pltpu.SemaphoreType.DMA,
        ],
    )(ys, pos0, pos1, G0, G1)


def kernel(x, wg, W_gate, b_gate, W_up, b_up, W_down, b_down):
    pos0, pos1, block_expert, nvalid, G0, G1, xp = _router(x, wg)
    xs = _dispatch(xp, pos0, pos1)
    ys = _experts(xs, W_gate, b_gate, W_up, b_up, W_down, b_down,
                  block_expert, nvalid)
    return _combine(ys, pos0, pos1, G0, G1)


# speedup vs baseline: 1.4788x; 1.0293x over previous
import jax
import jax.numpy as jnp
from jax import lax
from jax.experimental import pallas as pl
from jax.experimental.pallas import tpu as pltpu
from jax.experimental.pallas import tpu_sc as plsc

T = 2048
D = 1024
DFF = 2048
E = 8
K = 2

BT = 256
G_MAX = T * K // BT + E
N_PAD = G_MAX * BT

NC = 2
NS = 16
NW = NC * NS
TPW = T // NW
CH = 32


HMASK = -65536


def _pack_rows(y):
    half = y.shape[1] // 2
    a = y[:, :half].astype(jnp.bfloat16).astype(jnp.float32)
    b = y[:, half:].astype(jnp.bfloat16).astype(jnp.float32)
    ia = lax.bitcast_convert_type(a, jnp.int32)
    ib = lax.bitcast_convert_type(b, jnp.int32)
    return lax.shift_right_logical(ia, 16) | (ib & HMASK)


def _unpack_rows(w):
    a = lax.bitcast_convert_type(lax.shift_left(w, 16), jnp.float32)
    b = lax.bitcast_convert_type(w & HMASK, jnp.float32)
    return jnp.concatenate([a, b], axis=1).astype(jnp.bfloat16)


def _router_body(x_ref, wg_ref, p0_ref, p1_ref, be_ref, nv_ref,
                 g0_ref, g1_ref, xp_ref):
    xp_ref[...] = _pack_rows(x_ref[...])
    logits = jnp.dot(x_ref[...], wg_ref[...],
                     preferred_element_type=jnp.float32)
    m0 = jnp.max(logits, axis=1)
    i0 = jnp.argmax(logits, axis=1).astype(jnp.int32)
    lane = lax.broadcasted_iota(jnp.int32, (T, E), 1)
    masked = jnp.where(lane == i0[:, None], -jnp.inf, logits)
    m1 = jnp.max(masked, axis=1)
    i1 = jnp.argmax(masked, axis=1).astype(jnp.int32)
    r = jnp.exp(m1 - m0)
    g0_ref[...] = jnp.broadcast_to((1.0 / (1.0 + r))[:, None], (T, 128))
    g1_ref[...] = jnp.broadcast_to((r / (1.0 + r))[:, None], (T, 128))

    e_all = jnp.concatenate([i0, i1])
    erow = jnp.arange(E, dtype=jnp.int32)
    hot = (e_all[None, :] == erow[:, None]).astype(jnp.int32)
    csum = hot
    shift = 1
    while shift < T * K:
        z = jnp.zeros((E, shift), jnp.int32)
        csum = csum + jnp.concatenate([z, csum[:, :-shift]], axis=1)
        shift *= 2
    rank = jnp.sum(hot * csum, axis=0) - 1
    counts = csum[:, -1]
    nblk = (counts + BT - 1) // BT
    tri = (erow[:, None] <= erow[None, :]).astype(jnp.float32)
    cs = jnp.dot(nblk.astype(jnp.float32), tri,
                 preferred_element_type=jnp.float32).astype(jnp.int32)
    total = cs[E - 1]
    base_rows = (cs - nblk) * BT
    pos = jnp.sum(hot * base_rows[:, None], axis=0) + rank
    p0_ref[...] = pos[:T]
    p1_ref[...] = pos[T:]
    gcol = lax.broadcasted_iota(jnp.int32, (G_MAX, E), 0)
    csb = jnp.broadcast_to(cs[None, :], (G_MAX, E))
    be_ref[...] = jnp.sum(
        ((gcol >= csb) & (csb < total)).astype(jnp.int32), axis=1)
    nv_ref[...] = jnp.broadcast_to(total, (1,))


def _router(x, wg):
    return pl.pallas_call(
        _router_body,
        out_shape=(
            jax.ShapeDtypeStruct((T,), jnp.int32),
            jax.ShapeDtypeStruct((T,), jnp.int32),
            jax.ShapeDtypeStruct((G_MAX,), jnp.int32),
            jax.ShapeDtypeStruct((1,), jnp.int32),
            jax.ShapeDtypeStruct((T, 128), jnp.float32),
            jax.ShapeDtypeStruct((T, 128), jnp.float32),
            jax.ShapeDtypeStruct((T, D // 2), jnp.int32),
        ),
    )(x, wg)


def _dispatch_body(x_hbm, p0_hbm, p1_hbm, xs_hbm, xrows, idx0, idx1, sem):
    wid = lax.axis_index("s") * NC + lax.axis_index("c")
    base = wid * TPW
    pltpu.sync_copy(x_hbm.at[pl.ds(base, TPW)], xrows)
    pltpu.sync_copy(p0_hbm.at[pl.ds(base, TPW)], idx0)
    pltpu.sync_copy(p1_hbm.at[pl.ds(base, TPW)], idx1)
    c0 = pltpu.async_copy(xrows, xs_hbm.at[idx0], sem)
    c1 = pltpu.async_copy(xrows, xs_hbm.at[idx1], sem)
    c0.wait()
    c1.wait()


def _dispatch(x, pos0, pos1):
    mesh = plsc.VectorSubcoreMesh(core_axis_name="c", subcore_axis_name="s")
    return pl.kernel(
        _dispatch_body,
        out_type=jax.ShapeDtypeStruct((N_PAD, D // 2), jnp.int32),
        mesh=mesh,
        scratch_types=[
            pltpu.VMEM((TPW, D // 2), jnp.int32),
            pltpu.VMEM((TPW,), jnp.int32),
            pltpu.VMEM((TPW,), jnp.int32),
            pltpu.SemaphoreType.DMA,
        ],
    )(x, pos0, pos1)


def _gmm_body(be_ref, nv_ref, x_ref, wg_ref, bg_ref, wu_ref, bu_ref,
              wd_ref, bd_ref, o_ref):
    @pl.when(pl.program_id(0) < nv_ref[0])
    def _():
        xb = _unpack_rows(x_ref[...])
        h = jnp.dot(xb, wg_ref[0].astype(jnp.bfloat16),
                    preferred_element_type=jnp.float32) + bg_ref[0]
        u = jnp.dot(xb, wu_ref[0].astype(jnp.bfloat16),
                    preferred_element_type=jnp.float32) + bu_ref[0]
        act = (h * jax.nn.sigmoid(h) * u).astype(jnp.bfloat16)
        o_ref[...] = jnp.dot(act, wd_ref[0].astype(jnp.bfloat16),
                             preferred_element_type=jnp.float32) + bd_ref[0]


def _experts(xs, W_gate, b_gate, W_up, b_up, W_down, b_down,
             block_expert, nvalid):
    grid_spec = pltpu.PrefetchScalarGridSpec(
        num_scalar_prefetch=2,
        grid=(G_MAX,),
        in_specs=[
            pl.BlockSpec((BT, D // 2),
                         lambda g, be, nv: (jnp.minimum(g, nv[0] - 1), 0)),
            pl.BlockSpec((1, D, DFF), lambda g, be, nv: (be[g], 0, 0)),
            pl.BlockSpec((1, 1, DFF), lambda g, be, nv: (be[g], 0, 0)),
            pl.BlockSpec((1, D, DFF), lambda g, be, nv: (be[g], 0, 0)),
            pl.BlockSpec((1, 1, DFF), lambda g, be, nv: (be[g], 0, 0)),
            pl.BlockSpec((1, DFF, D), lambda g, be, nv: (be[g], 0, 0)),
            pl.BlockSpec((1, 1, D), lambda g, be, nv: (be[g], 0, 0)),
        ],
        out_specs=pl.BlockSpec((BT, D),
                               lambda g, be, nv: (jnp.minimum(g, nv[0]), 0)),
    )
    return pl.pallas_call(
        _gmm_body,
        grid_spec=grid_spec,
        out_shape=jax.ShapeDtypeStruct((N_PAD, D), jnp.float32),
        compiler_params=pltpu.CompilerParams(
            dimension_semantics=("arbitrary",),
            vmem_limit_bytes=128 * 1024 * 1024,
        ),
    )(block_expert, nvalid, xs,
      W_gate, b_gate.reshape(E, 1, DFF),
      W_up, b_up.reshape(E, 1, DFF),
      W_down, b_down.reshape(E, 1, D))


def _combine_body(ys_hbm, p0_hbm, p1_hbm, g0_hbm, g1_hbm, y_hbm,
                  rows0, rows1, yout, g0v, g1v, idx0, idx1, sem):
    wid = lax.axis_index("s") * NC + lax.axis_index("c")
    half = D // 2
    for c in range(TPW // CH):
        base = wid * TPW + c * CH
        pltpu.sync_copy(p0_hbm.at[pl.ds(base, CH)], idx0)
        pltpu.sync_copy(p1_hbm.at[pl.ds(base, CH)], idx1)
        pltpu.sync_copy(g0_hbm.at[pl.ds(base, CH)], g0v)
        pltpu.sync_copy(g1_hbm.at[pl.ds(base, CH)], g1v)
        c0 = pltpu.async_copy(ys_hbm.at[idx0], rows0, sem)
        c1 = pltpu.async_copy(ys_hbm.at[idx1], rows1, sem)
        c0.wait()
        c1.wait()
        for j in range(CH):
            ga = g0v[j, 0:16]
            gb = g1v[j, 0:16]

            def body(i, _, ga=ga, gb=gb, j=j):
                for u in range(8):
                    s = pl.ds(i * 128 + u * 16, 16)
                    yout[j, s] = ga * rows0[j, s] + gb * rows1[j, s]
                return 0

            lax.fori_loop(0, D // 128, body, 0)
        pltpu.sync_copy(yout, y_hbm.at[pl.ds(base, CH)])


def _combine(ys, pos0, pos1, G0, G1):
    mesh = plsc.VectorSubcoreMesh(core_axis_name="c", subcore_axis_name="s")
    return pl.kernel(
        _combine_body,
        out_type=jax.ShapeDtypeStruct((T, D), jnp.float32),
        mesh=mesh,
        compiler_params=pltpu.CompilerParams(needs_layout_passes=False),
        scratch_types=[
            pltpu.VMEM((CH, D), jnp.float32),
            pltpu.VMEM((CH, D), jnp.float32),
            pltpu.VMEM((CH, D), jnp.float32),
            pltpu.VMEM((CH, 128), jnp.float32),
            pltpu.VMEM((CH, 128), jnp.float32),
            pltpu.VMEM((CH,), jnp.int32),
            pltpu.VMEM((CH,), jnp.int32),
            pltpu.SemaphoreType.DMA,
        ],
    )(ys, pos0, pos1, G0, G1)


def kernel(x, wg, W_gate, b_gate, W_up, b_up, W_down, b_down):
    pos0, pos1, block_expert, nvalid, G0, G1, xp = _router(x, wg)
    xs = _dispatch(xp, pos0, pos1)
    ys = _experts(xs, W_gate, b_gate, W_up, b_up, W_down, b_down,
                  block_expert, nvalid)
    return _combine(ys, pos0, pos1, G0, G1)
